# Initial kernel scaffold; baseline (speedup 1.0000x reference)
import functools

import jax
import jax.numpy as jnp
from jax import lax
from jax.experimental import pallas as pl
from jax.experimental.pallas import tpu as pltpu
from jax.experimental.pallas import tpu_sc as plsc

N = 10000
F = 128
NSUB = 16
K = 128
CH = 157
EPAD = NSUB * CH * K
NPAD = 10112
RPT = NPAD // NSUB
DUMMY = N
SLICES = ((0, 128), (128, 128), (256, 128), (384, 128), (512, 120))
PAIRS = (CH - 1) // 2


def _mesh():
    return plsc.VectorSubcoreMesh(core_axis_name="c", subcore_axis_name="s")


def _sc_degree(dst0, dst1, ones16, zeros16):
    @functools.partial(
        pl.kernel,
        out_type=(
            jax.ShapeDtypeStruct((NPAD, 16), jnp.float32),
            jax.ShapeDtypeStruct((NPAD, 16), jnp.float32),
        ),
        mesh=_mesh(),
        scratch_types=[
            pltpu.VMEM((CH, K), jnp.int32),
            pltpu.VMEM((K, 16), jnp.float32),
            pltpu.VMEM((RPT, 16), jnp.float32),
            pltpu.VMEM_SHARED((NPAD, 16), jnp.float32),
        ],
    )
    def k(d0_h, d1_h, ones_h, zeros_h, deg0_h, deg1_h, dst_v, ones_v, bnc_v, acc):
        c = lax.axis_index("c")
        s = lax.axis_index("s")
        base = s * RPT
        pltpu.sync_copy(ones_h, ones_v)
        pltpu.sync_copy(zeros_h, bnc_v)
        pltpu.sync_copy(bnc_v, acc.at[pl.ds(base, RPT)])

        @pl.when(c == 0)
        def _():
            pltpu.sync_copy(d0_h.at[s], dst_v)

        @pl.when(c == 1)
        def _():
            pltpu.sync_copy(d1_h.at[s], dst_v)

        plsc.subcore_barrier()

        @pl.loop(0, CH)
        def _(j):
            pltpu.sync_copy(ones_v, acc.at[dst_v.at[j]], add=True)

        plsc.subcore_barrier()
        pltpu.sync_copy(acc.at[pl.ds(base, RPT)], bnc_v)

        @pl.when(c == 0)
        def _():
            pltpu.sync_copy(bnc_v, deg0_h.at[pl.ds(base, RPT)])

        @pl.when(c == 1)
        def _():
            pltpu.sync_copy(bnc_v, deg1_h.at[pl.ds(base, RPT)])

    return k(dst0, dst1, ones16, zeros16)


def _sc_scatter(table0, table1, src0, dst0, src1, dst1, zeros128):
    @functools.partial(
        pl.kernel,
        out_type=(
            jax.ShapeDtypeStruct((NPAD, F), jnp.float32),
            jax.ShapeDtypeStruct((NPAD, F), jnp.float32),
        ),
        mesh=_mesh(),
        scratch_types=[
            pltpu.VMEM((CH, K), jnp.int32),
            pltpu.VMEM((CH, K), jnp.int32),
            pltpu.VMEM((K, F), jnp.float32),
            pltpu.VMEM((K, F), jnp.float32),
            pltpu.VMEM((K, F), jnp.float32),
            pltpu.VMEM_SHARED((NPAD, F), jnp.float32),
            pltpu.SemaphoreType.DMA,
            pltpu.SemaphoreType.DMA,
        ],
    )
    def k(t0, t1, s0, d0, s1, d1, z_h, o0, o1,
          src_v, dst_v, ra, rb, zv, acc, sema, semb):
        c = lax.axis_index("c")
        s = lax.axis_index("s")
        base = s * RPT

        pltpu.sync_copy(z_h, zv)
        for off, sz in SLICES:
            pltpu.sync_copy(zv.at[pl.ds(0, sz)], acc.at[pl.ds(base + off, sz)])

        @pl.when(c == 0)
        def _():
            pltpu.sync_copy(s0.at[s], src_v)
            pltpu.sync_copy(d0.at[s], dst_v)

        @pl.when(c == 1)
        def _():
            pltpu.sync_copy(s1.at[s], src_v)
            pltpu.sync_copy(d1.at[s], dst_v)

        plsc.subcore_barrier()

        def do_graph(tab):
            pltpu.async_copy(tab.at[src_v.at[0]], ra, sema)

            @pl.loop(0, PAIRS)
            def _(p):
                j = 2 * p
                pltpu.async_copy(tab.at[src_v.at[j + 1]], rb, semb)
                pltpu.make_async_copy(tab.at[src_v.at[j]], ra, sema).wait()
                pltpu.sync_copy(ra, acc.at[dst_v.at[j]], add=True)
                pltpu.async_copy(tab.at[src_v.at[j + 2]], ra, sema)
                pltpu.make_async_copy(tab.at[src_v.at[j + 1]], rb, semb).wait()
                pltpu.sync_copy(rb, acc.at[dst_v.at[j + 1]], add=True)

            pltpu.make_async_copy(tab.at[src_v.at[CH - 1]], ra, sema).wait()
            pltpu.sync_copy(ra, acc.at[dst_v.at[CH - 1]], add=True)

        @pl.when(c == 0)
        def _():
            do_graph(t0)

        @pl.when(c == 1)
        def _():
            do_graph(t1)

        plsc.subcore_barrier()

        def copy_out(o):
            for off, sz in SLICES:
                pltpu.sync_copy(acc.at[pl.ds(base + off, sz)], zv.at[pl.ds(0, sz)])
                pltpu.sync_copy(zv.at[pl.ds(0, sz)], o.at[pl.ds(base + off, sz)])

        @pl.when(c == 0)
        def _():
            copy_out(o0)

        @pl.when(c == 1)
        def _():
            copy_out(o1)

    return k(table0, table1, src0, dst0, src1, dst1, zeros128)


def _tc_scale_matmul(x, deg, W):
    def body(x_ref, d_ref, w_ref, y_ref):
        d = d_ref[:, 0:1]
        norm = lax.rsqrt(jnp.maximum(d, 1.0))
        y_ref[...] = jnp.dot(x_ref[...], w_ref[...],
                             preferred_element_type=jnp.float32) * norm

    return pl.pallas_call(
        body,
        grid=(NPAD // 128,),
        in_specs=[
            pl.BlockSpec((128, F), lambda i: (i, 0)),
            pl.BlockSpec((128, 16), lambda i: (i, 0)),
            pl.BlockSpec((F, F), lambda i: (0, 0)),
        ],
        out_specs=pl.BlockSpec((128, F), lambda i: (i, 0)),
        out_shape=jax.ShapeDtypeStruct((NPAD, F), jnp.float32),
    )(x, deg, W)


def _tc_mid(agg, deg, b, W):
    def body(a_ref, d_ref, b_ref, w_ref, y_ref):
        i = pl.program_id(0)
        d = d_ref[:, 0:1]
        norm = lax.rsqrt(jnp.maximum(d, 1.0))
        h = jnp.maximum(a_ref[...] * norm + b_ref[...], 0.0)
        row = lax.broadcasted_iota(jnp.int32, (128, 1), 0) + i * 128
        h = jnp.where(row < N, h, 0.0)
        y_ref[...] = jnp.dot(h, w_ref[...],
                             preferred_element_type=jnp.float32) * norm

    return pl.pallas_call(
        body,
        grid=(NPAD // 128,),
        in_specs=[
            pl.BlockSpec((128, F), lambda i: (i, 0)),
            pl.BlockSpec((128, 16), lambda i: (i, 0)),
            pl.BlockSpec((1, F), lambda i: (0, 0)),
            pl.BlockSpec((F, F), lambda i: (0, 0)),
        ],
        out_specs=pl.BlockSpec((128, F), lambda i: (i, 0)),
        out_shape=jax.ShapeDtypeStruct((NPAD, F), jnp.float32),
    )(agg, deg, b, W)


def _tc_final(agg, deg, b):
    def body(a_ref, d_ref, b_ref, y_ref):
        d = d_ref[:, 0:1]
        norm = lax.rsqrt(jnp.maximum(d, 1.0))
        y_ref[...] = a_ref[...] * norm + b_ref[...]

    return pl.pallas_call(
        body,
        grid=(NPAD // 128,),
        in_specs=[
            pl.BlockSpec((128, F), lambda i: (i, 0)),
            pl.BlockSpec((128, 16), lambda i: (i, 0)),
            pl.BlockSpec((1, F), lambda i: (0, 0)),
        ],
        out_specs=pl.BlockSpec((128, F), lambda i: (i, 0)),
        out_shape=jax.ShapeDtypeStruct((NPAD, F), jnp.float32),
    )(agg, deg, b)


def kernel(API_x, API_edge_index, skill_x, skill_edge_index, W1, b1, W2, b2):
    f32 = jnp.float32

    def pad_x(x):
        return jnp.zeros((NPAD, F), f32).at[:N].set(x.astype(f32))

    def prep_edges(ei):
        e = ei.astype(jnp.int32)
        pad = jnp.full((EPAD - e.shape[1],), DUMMY, jnp.int32)
        src = jnp.concatenate([e[0], pad]).reshape(NSUB, CH, K)
        dst = jnp.concatenate([e[1], pad]).reshape(NSUB, CH, K)
        return src, dst

    x0 = pad_x(API_x)
    x1 = pad_x(skill_x)
    s0, d0 = prep_edges(API_edge_index)
    s1, d1 = prep_edges(skill_edge_index)

    ones16 = jnp.ones((K, 16), f32)
    zeros16 = jnp.zeros((RPT, 16), f32)
    zeros128 = jnp.zeros((K, F), f32)
    b1r = b1.reshape(1, F).astype(f32)
    b2r = b2.reshape(1, F).astype(f32)

    deg0, deg1 = _sc_degree(d0, d1, ones16, zeros16)

    y0 = _tc_scale_matmul(x0, deg0, W1)
    y1 = _tc_scale_matmul(x1, deg1, W1)

    a0, a1 = _sc_scatter(y0, y1, s0, d0, s1, d1, zeros128)

    z0 = _tc_mid(a0, deg0, b1r, W2)
    z1 = _tc_mid(a1, deg1, b1r, W2)

    g0, g1 = _sc_scatter(z0, z1, s0, d0, s1, d1, zeros128)

    out0 = _tc_final(g0, deg0, b2r)
    out1 = _tc_final(g1, deg1, b2r)

    return (out0[:N], out1[:N])


# speedup vs baseline: 3.4320x; 3.4320x over previous
import functools

import jax
import jax.numpy as jnp
from jax import lax
from jax.experimental import pallas as pl
from jax.experimental.pallas import tpu as pltpu
from jax.experimental.pallas import tpu_sc as plsc

N = 10000
F = 128
NSUB = 16
K = 128
CH = 160
EPAD = NSUB * CH * K
NPAD = 10112
RPT = NPAD // NSUB
DUMMY = N
OFFS = (0, 128, 256, 384, 504)
NZC = len(OFFS)


def _mesh():
    return plsc.VectorSubcoreMesh(core_axis_name="c", subcore_axis_name="s")


def _sc_edge_pass(t0a, t1a, src0, dst0, src1, dst1,
                  fill_h, zeros_h, iota_h, with_gather):
    @functools.partial(
        pl.kernel,
        out_type=(
            jax.ShapeDtypeStruct((NPAD, F), jnp.float32),
            jax.ShapeDtypeStruct((NPAD, F), jnp.float32),
        ),
        mesh=_mesh(),
        scratch_types=[
            pltpu.VMEM((K,), jnp.int32),
            pltpu.VMEM((K,), jnp.int32),
            pltpu.VMEM((K,), jnp.int32),
            pltpu.VMEM((K, F), jnp.float32),
            pltpu.VMEM((K, F), jnp.float32),
            pltpu.VMEM_SHARED((NPAD, F), jnp.float32),
            pltpu.SemaphoreType.DMA,
        ],
    )
    def k(t0, t1, s0, d0, s1, d1, fill, zeros, iota, o0, o1,
          sbuf, dbuf, izbuf, ra, rb, acc, sema):
        c = lax.axis_index("c")
        s = lax.axis_index("s")
        base = s * RPT

        pltpu.sync_copy(zeros, rb)

        @pl.loop(0, NZC)
        def _(q):
            pltpu.sync_copy(iota.at[s * NZC + q], izbuf)
            pltpu.sync_copy(rb, acc.at[izbuf])

        plsc.subcore_barrier()

        if not with_gather:
            pltpu.sync_copy(fill, ra)

        def do_graph(tab, src_h, dst_h):
            @pl.loop(0, CH)
            def _(j):
                pltpu.sync_copy(dst_h.at[s * CH + j], dbuf)
                if with_gather:
                    pltpu.sync_copy(src_h.at[s * CH + j], sbuf)
                    pltpu.async_copy(tab.at[sbuf], ra, sema).wait()
                pltpu.sync_copy(ra, acc.at[dbuf], add=True)

        @pl.when(c == 0)
        def _():
            do_graph(t0, s0, d0)

        @pl.when(c == 1)
        def _():
            do_graph(t1, s1, d1)

        plsc.subcore_barrier()

        def copy_out(o):
            for q, off in enumerate(OFFS):
                pltpu.sync_copy(iota.at[s * NZC + q], izbuf)
                pltpu.async_copy(acc.at[izbuf], rb, sema).wait()
                pltpu.sync_copy(rb, o.at[pl.ds(base + off, K)])

        @pl.when(c == 0)
        def _():
            copy_out(o0)

        @pl.when(c == 1)
        def _():
            copy_out(o1)

    return k(t0a, t1a, src0, dst0, src1, dst1, fill_h, zeros_h, iota_h)


def _tc_scale_matmul(x, deg, W):
    def body(x_ref, d_ref, w_ref, y_ref):
        d = d_ref[:, 0:1]
        norm = lax.rsqrt(jnp.maximum(d, 1.0))
        y_ref[...] = jnp.dot(x_ref[...], w_ref[...],
                             preferred_element_type=jnp.float32) * norm

    return pl.pallas_call(
        body,
        grid=(NPAD // 128,),
        in_specs=[
            pl.BlockSpec((128, F), lambda i: (i, 0)),
            pl.BlockSpec((128, F), lambda i: (i, 0)),
            pl.BlockSpec((F, F), lambda i: (0, 0)),
        ],
        out_specs=pl.BlockSpec((128, F), lambda i: (i, 0)),
        out_shape=jax.ShapeDtypeStruct((NPAD, F), jnp.float32),
    )(x, deg, W)


def _tc_mid(agg, deg, b, W):
    def body(a_ref, d_ref, b_ref, w_ref, y_ref):
        i = pl.program_id(0)
        d = d_ref[:, 0:1]
        norm = lax.rsqrt(jnp.maximum(d, 1.0))
        h = jnp.maximum(a_ref[...] * norm + b_ref[...], 0.0)
        row = lax.broadcasted_iota(jnp.int32, (128, 1), 0) + i * 128
        h = jnp.where(row < N, h, 0.0)
        y_ref[...] = jnp.dot(h, w_ref[...],
                             preferred_element_type=jnp.float32) * norm

    return pl.pallas_call(
        body,
        grid=(NPAD // 128,),
        in_specs=[
            pl.BlockSpec((128, F), lambda i: (i, 0)),
            pl.BlockSpec((128, F), lambda i: (i, 0)),
            pl.BlockSpec((1, F), lambda i: (0, 0)),
            pl.BlockSpec((F, F), lambda i: (0, 0)),
        ],
        out_specs=pl.BlockSpec((128, F), lambda i: (i, 0)),
        out_shape=jax.ShapeDtypeStruct((NPAD, F), jnp.float32),
    )(agg, deg, b, W)


def _tc_final(agg, deg, b):
    def body(a_ref, d_ref, b_ref, y_ref):
        d = d_ref[:, 0:1]
        norm = lax.rsqrt(jnp.maximum(d, 1.0))
        y_ref[...] = a_ref[...] * norm + b_ref[...]

    return pl.pallas_call(
        body,
        grid=(NPAD // 128,),
        in_specs=[
            pl.BlockSpec((128, F), lambda i: (i, 0)),
            pl.BlockSpec((128, F), lambda i: (i, 0)),
            pl.BlockSpec((1, F), lambda i: (0, 0)),
        ],
        out_specs=pl.BlockSpec((128, F), lambda i: (i, 0)),
        out_shape=jax.ShapeDtypeStruct((NPAD, F), jnp.float32),
    )(agg, deg, b)


def kernel(API_x, API_edge_index, skill_x, skill_edge_index, W1, b1, W2, b2):
    f32 = jnp.float32

    def pad_x(x):
        return jnp.zeros((NPAD, F), f32).at[:N].set(x.astype(f32))

    def prep_edges(ei):
        e = ei.astype(jnp.int32)
        pad = jnp.full((EPAD - e.shape[1],), DUMMY, jnp.int32)
        src = jnp.concatenate([e[0], pad]).reshape(NSUB * CH, K)
        dst = jnp.concatenate([e[1], pad]).reshape(NSUB * CH, K)
        return src, dst

    x0 = pad_x(API_x)
    x1 = pad_x(skill_x)
    s0, d0 = prep_edges(API_edge_index)
    s1, d1 = prep_edges(skill_edge_index)

    ones_kf = jnp.ones((K, F), f32)
    zeros_kf = jnp.zeros((K, F), f32)
    iota_h = (
        jnp.arange(NSUB, dtype=jnp.int32)[:, None, None] * RPT
        + jnp.asarray(OFFS, jnp.int32)[None, :, None]
        + jnp.arange(K, dtype=jnp.int32)[None, None, :]
    ).reshape(NSUB * NZC, K)
    b1r = b1.reshape(1, F).astype(f32)
    b2r = b2.reshape(1, F).astype(f32)

    deg0, deg1 = _sc_edge_pass(zeros_kf, zeros_kf, s0, d0, s1, d1,
                               ones_kf, zeros_kf, iota_h, with_gather=False)

    y0 = _tc_scale_matmul(x0, deg0, W1)
    y1 = _tc_scale_matmul(x1, deg1, W1)

    a0, a1 = _sc_edge_pass(y0, y1, s0, d0, s1, d1,
                           zeros_kf, zeros_kf, iota_h, with_gather=True)

    z0 = _tc_mid(a0, deg0, b1r, W2)
    z1 = _tc_mid(a1, deg1, b1r, W2)

    g0, g1 = _sc_edge_pass(z0, z1, s0, d0, s1, d1,
                           zeros_kf, zeros_kf, iota_h, with_gather=True)

    out0 = _tc_final(g0, deg0, b2r)
    out1 = _tc_final(g1, deg1, b2r)

    return (out0[:N], out1[:N])


# speedup vs baseline: 4.2519x; 1.2389x over previous
import functools

import jax
import jax.numpy as jnp
from jax import lax
from jax.experimental import pallas as pl
from jax.experimental.pallas import tpu as pltpu
from jax.experimental.pallas import tpu_sc as plsc

N = 10000
F = 128
NSUB = 16
K = 128
CH = 160
EPAD = NSUB * CH * K
NPAD = 10112
RPT = NPAD // NSUB
DUMMY = N
OFFS = (0, 128, 256, 384, 504)
NZC = len(OFFS)


def _mesh():
    return plsc.VectorSubcoreMesh(core_axis_name="c", subcore_axis_name="s")


def _sc_edge_pass(t0a, t1a, src0, dst0, src1, dst1,
                  fill_h, zeros_h, iota_h, with_gather):
    @functools.partial(
        pl.kernel,
        out_type=(
            jax.ShapeDtypeStruct((NPAD, F), jnp.float32),
            jax.ShapeDtypeStruct((NPAD, F), jnp.float32),
        ),
        mesh=_mesh(),
        scratch_types=[
            pltpu.VMEM((K,), jnp.int32),
            pltpu.VMEM((K,), jnp.int32),
            pltpu.VMEM((K,), jnp.int32),
            pltpu.VMEM((K,), jnp.int32),
            pltpu.VMEM((K,), jnp.int32),
            pltpu.VMEM((K, F), jnp.float32),
            pltpu.VMEM((K, F), jnp.float32),
            pltpu.VMEM_SHARED((NPAD, F), jnp.float32),
            pltpu.SemaphoreType.DMA,
            pltpu.SemaphoreType.DMA,
        ],
    )
    def k(t0, t1, s0, d0, s1, d1, fill, zeros, iota, o0, o1,
          sbuf, sbuf1, dbuf, dbuf1, izbuf, ra, rb, acc, sema, semb):
        c = lax.axis_index("c")
        s = lax.axis_index("s")
        base = s * RPT

        pltpu.sync_copy(zeros, rb)

        @pl.loop(0, NZC)
        def _(q):
            pltpu.sync_copy(iota.at[s * NZC + q], izbuf)
            pltpu.sync_copy(rb, acc.at[izbuf])

        plsc.subcore_barrier()

        if not with_gather:
            pltpu.sync_copy(fill, ra)

        def do_graph(tab, src_h, dst_h):
            if not with_gather:
                @pl.loop(0, CH)
                def _(j):
                    pltpu.sync_copy(dst_h.at[s * CH + j], dbuf)
                    pltpu.sync_copy(ra, acc.at[dbuf], add=True)
                return

            c0 = s * CH
            pltpu.sync_copy(src_h.at[c0], sbuf)
            pltpu.sync_copy(dst_h.at[c0], dbuf)
            pltpu.async_copy(tab.at[sbuf], ra, sema)

            @pl.loop(0, CH // 2)
            def _(p):
                j = c0 + 2 * p
                pltpu.sync_copy(src_h.at[j + 1], sbuf1)
                pltpu.sync_copy(dst_h.at[j + 1], dbuf1)
                pltpu.async_copy(tab.at[sbuf1], rb, semb)
                pltpu.make_async_copy(tab.at[sbuf], ra, sema).wait()
                pltpu.sync_copy(ra, acc.at[dbuf], add=True)

                @pl.when(p < CH // 2 - 1)
                def _():
                    pltpu.sync_copy(src_h.at[j + 2], sbuf)
                    pltpu.sync_copy(dst_h.at[j + 2], dbuf)
                    pltpu.async_copy(tab.at[sbuf], ra, sema)

                pltpu.make_async_copy(tab.at[sbuf1], rb, semb).wait()
                pltpu.sync_copy(rb, acc.at[dbuf1], add=True)

        @pl.when(c == 0)
        def _():
            do_graph(t0, s0, d0)

        @pl.when(c == 1)
        def _():
            do_graph(t1, s1, d1)

        plsc.subcore_barrier()

        def copy_out(o):
            for q, off in enumerate(OFFS):
                pltpu.sync_copy(iota.at[s * NZC + q], izbuf)
                pltpu.async_copy(acc.at[izbuf], rb, sema).wait()
                pltpu.sync_copy(rb, o.at[pl.ds(base + off, K)])

        @pl.when(c == 0)
        def _():
            copy_out(o0)

        @pl.when(c == 1)
        def _():
            copy_out(o1)

    return k(t0a, t1a, src0, dst0, src1, dst1, fill_h, zeros_h, iota_h)


def _tc_scale_matmul(x, deg, W):
    def body(x_ref, d_ref, w_ref, y_ref):
        d = d_ref[:, 0:1]
        norm = lax.rsqrt(jnp.maximum(d, 1.0))
        y_ref[...] = jnp.dot(x_ref[...], w_ref[...],
                             preferred_element_type=jnp.float32) * norm

    return pl.pallas_call(
        body,
        grid=(NPAD // 128,),
        in_specs=[
            pl.BlockSpec((128, F), lambda i: (i, 0)),
            pl.BlockSpec((128, F), lambda i: (i, 0)),
            pl.BlockSpec((F, F), lambda i: (0, 0)),
        ],
        out_specs=pl.BlockSpec((128, F), lambda i: (i, 0)),
        out_shape=jax.ShapeDtypeStruct((NPAD, F), jnp.float32),
    )(x, deg, W)


def _tc_mid(agg, deg, b, W):
    def body(a_ref, d_ref, b_ref, w_ref, y_ref):
        i = pl.program_id(0)
        d = d_ref[:, 0:1]
        norm = lax.rsqrt(jnp.maximum(d, 1.0))
        h = jnp.maximum(a_ref[...] * norm + b_ref[...], 0.0)
        row = lax.broadcasted_iota(jnp.int32, (128, 1), 0) + i * 128
        h = jnp.where(row < N, h, 0.0)
        y_ref[...] = jnp.dot(h, w_ref[...],
                             preferred_element_type=jnp.float32) * norm

    return pl.pallas_call(
        body,
        grid=(NPAD // 128,),
        in_specs=[
            pl.BlockSpec((128, F), lambda i: (i, 0)),
            pl.BlockSpec((128, F), lambda i: (i, 0)),
            pl.BlockSpec((1, F), lambda i: (0, 0)),
            pl.BlockSpec((F, F), lambda i: (0, 0)),
        ],
        out_specs=pl.BlockSpec((128, F), lambda i: (i, 0)),
        out_shape=jax.ShapeDtypeStruct((NPAD, F), jnp.float32),
    )(agg, deg, b, W)


def _tc_final(agg, deg, b):
    def body(a_ref, d_ref, b_ref, y_ref):
        d = d_ref[:, 0:1]
        norm = lax.rsqrt(jnp.maximum(d, 1.0))
        y_ref[...] = a_ref[...] * norm + b_ref[...]

    return pl.pallas_call(
        body,
        grid=(NPAD // 128,),
        in_specs=[
            pl.BlockSpec((128, F), lambda i: (i, 0)),
            pl.BlockSpec((128, F), lambda i: (i, 0)),
            pl.BlockSpec((1, F), lambda i: (0, 0)),
        ],
        out_specs=pl.BlockSpec((128, F), lambda i: (i, 0)),
        out_shape=jax.ShapeDtypeStruct((NPAD, F), jnp.float32),
    )(agg, deg, b)


def kernel(API_x, API_edge_index, skill_x, skill_edge_index, W1, b1, W2, b2):
    f32 = jnp.float32

    def pad_x(x):
        return jnp.zeros((NPAD, F), f32).at[:N].set(x.astype(f32))

    def prep_edges(ei):
        e = ei.astype(jnp.int32)
        pad = jnp.full((EPAD - e.shape[1],), DUMMY, jnp.int32)
        src = jnp.concatenate([e[0], pad]).reshape(NSUB * CH, K)
        dst = jnp.concatenate([e[1], pad]).reshape(NSUB * CH, K)
        return src, dst

    x0 = pad_x(API_x)
    x1 = pad_x(skill_x)
    s0, d0 = prep_edges(API_edge_index)
    s1, d1 = prep_edges(skill_edge_index)

    ones_kf = jnp.ones((K, F), f32)
    zeros_kf = jnp.zeros((K, F), f32)
    iota_h = (
        jnp.arange(NSUB, dtype=jnp.int32)[:, None, None] * RPT
        + jnp.asarray(OFFS, jnp.int32)[None, :, None]
        + jnp.arange(K, dtype=jnp.int32)[None, None, :]
    ).reshape(NSUB * NZC, K)
    b1r = b1.reshape(1, F).astype(f32)
    b2r = b2.reshape(1, F).astype(f32)

    deg0, deg1 = _sc_edge_pass(zeros_kf, zeros_kf, s0, d0, s1, d1,
                               ones_kf, zeros_kf, iota_h, with_gather=False)

    y0 = _tc_scale_matmul(x0, deg0, W1)
    y1 = _tc_scale_matmul(x1, deg1, W1)

    a0, a1 = _sc_edge_pass(y0, y1, s0, d0, s1, d1,
                           zeros_kf, zeros_kf, iota_h, with_gather=True)

    z0 = _tc_mid(a0, deg0, b1r, W2)
    z1 = _tc_mid(a1, deg1, b1r, W2)

    g0, g1 = _sc_edge_pass(z0, z1, s0, d0, s1, d1,
                           zeros_kf, zeros_kf, iota_h, with_gather=True)

    out0 = _tc_final(g0, deg0, b2r)
    out1 = _tc_final(g1, deg1, b2r)

    return (out0[:N], out1[:N])
